# initial kernel scaffold (unmeasured)
import jax
import jax.numpy as jnp
from jax import lax
from jax.experimental import pallas as pl
from jax.experimental.pallas import tpu as pltpu

N_DEV = 32
N_TOK = 512
D_IN = 256
D_OUT = 512
N_EXP = 128
E_LOCAL = N_EXP // N_DEV
ROWS = N_TOK // N_DEV


def kernel(x, router_W, route_idx, expert_W):
    def body(x_ref, rw_ref, idx_ref, ew_ref, out_ref,
             stage1_ref, comm1_ref, stage2_ref, comm2_ref,
             send1_sems, recv1_sems, send2_sems, recv2_sems):
        my = lax.axis_index("i")

        xf = x_ref[:, :]
        scores = jnp.dot(xf, rw_ref[:, :], preferred_element_type=jnp.float32)
        e_iota = lax.broadcasted_iota(jnp.int32, (N_TOK, N_EXP), 1)
        r0 = idx_ref[:, 0:1]
        r1 = idx_ref[:, 1:2]
        s0 = jnp.sum(jnp.where(e_iota == r0, scores, 0.0), axis=1, keepdims=True)
        s1 = jnp.sum(jnp.where(e_iota == r1, scores, 0.0), axis=1, keepdims=True)
        w0 = 1.0 / (1.0 + jnp.exp(s1 - s0))
        w1 = 1.0 - w0

        le = my * E_LOCAL + lax.broadcasted_iota(jnp.int32, (1, E_LOCAL), 1)
        wloc = (w0 * (r0 == le).astype(jnp.float32)
                + w1 * (r1 == le).astype(jnp.float32))
        partial = jnp.zeros((N_TOK, D_OUT), jnp.float32)
        for k in range(E_LOCAL):
            xk = (xf * wloc[:, k:k + 1]).astype(jnp.bfloat16)
            wk = ew_ref[k, :, :].astype(jnp.bfloat16)
            partial = partial + jnp.dot(xk, wk, preferred_element_type=jnp.float32)
        stage1_ref[:, :] = partial.astype(jnp.bfloat16)

        sends1 = []
        for d in range(1, N_DEV):
            dst = lax.rem(my + d, N_DEV)
            r = pltpu.make_async_remote_copy(
                src_ref=stage1_ref.at[pl.ds(dst * ROWS, ROWS), :],
                dst_ref=comm1_ref.at[my],
                send_sem=send1_sems.at[d],
                recv_sem=recv1_sems.at[my],
                device_id=(dst,),
                device_id_type=pl.DeviceIdType.MESH,
            )
            r.start()
            sends1.append(r)
        comm1_ref[pl.ds(my, 1)] = stage1_ref[pl.ds(my * ROWS, ROWS), :].reshape(
            1, ROWS, D_OUT)
        for d in range(1, N_DEV):
            src = lax.rem(my - d + N_DEV, N_DEV)
            rr = pltpu.make_async_remote_copy(
                src_ref=stage1_ref.at[pl.ds(0, ROWS), :],
                dst_ref=comm1_ref.at[src],
                send_sem=send1_sems.at[d],
                recv_sem=recv1_sems.at[src],
                device_id=(src,),
                device_id_type=pl.DeviceIdType.MESH,
            )
            rr.wait_recv()
        reduced = jnp.sum(comm1_ref[:, :, :].astype(jnp.float32), axis=0)
        for r in sends1:
            r.wait_send()

        stage2_ref[:, :] = reduced.astype(jnp.bfloat16)
        sends2 = []
        for d in range(1, N_DEV):
            dst = lax.rem(my + d, N_DEV)
            r = pltpu.make_async_remote_copy(
                src_ref=stage2_ref,
                dst_ref=comm2_ref.at[my],
                send_sem=send2_sems.at[d],
                recv_sem=recv2_sems.at[my],
                device_id=(dst,),
                device_id_type=pl.DeviceIdType.MESH,
            )
            r.start()
            sends2.append(r)
        comm2_ref[pl.ds(my, 1)] = stage2_ref[:, :].reshape(1, ROWS, D_OUT)
        for d in range(1, N_DEV):
            src = lax.rem(my - d + N_DEV, N_DEV)
            rr = pltpu.make_async_remote_copy(
                src_ref=stage2_ref,
                dst_ref=comm2_ref.at[src],
                send_sem=send2_sems.at[d],
                recv_sem=recv2_sems.at[src],
                device_id=(src,),
                device_id_type=pl.DeviceIdType.MESH,
            )
            rr.wait_recv()
        for r in sends2:
            r.wait_send()
        out_ref[:, :] = comm2_ref[:, :, :].astype(jnp.float32).reshape(N_TOK, D_OUT)

    return pl.pallas_call(
        body,
        out_shape=jax.ShapeDtypeStruct((N_TOK, D_OUT), jnp.float32),
        in_specs=[pl.BlockSpec(memory_space=pltpu.VMEM)] * 4,
        out_specs=pl.BlockSpec(memory_space=pltpu.VMEM),
        scratch_shapes=[
            pltpu.VMEM((N_TOK, D_OUT), jnp.bfloat16),
            pltpu.VMEM((N_DEV, ROWS, D_OUT), jnp.bfloat16),
            pltpu.VMEM((ROWS, D_OUT), jnp.bfloat16),
            pltpu.VMEM((N_DEV, ROWS, D_OUT), jnp.bfloat16),
            pltpu.SemaphoreType.DMA((N_DEV,)),
            pltpu.SemaphoreType.DMA((N_DEV,)),
            pltpu.SemaphoreType.DMA((N_DEV,)),
            pltpu.SemaphoreType.DMA((N_DEV,)),
        ],
        compiler_params=pltpu.CompilerParams(collective_id=0),
    )(x, router_W, route_idx, expert_W)


# baseline (device time: 37217 ns/iter reference)
import jax
import jax.numpy as jnp
from jax import lax
from jax.experimental import pallas as pl
from jax.experimental.pallas import tpu as pltpu

N_DEV = 32
N_TOK = 512
D_IN = 256
D_OUT = 512
N_EXP = 128
E_LOCAL = N_EXP // N_DEV
ROWS = N_TOK // N_DEV


def kernel(x, router_W, route_idx, expert_W):
    def body(x_ref, rw_ref, idx_ref, ew_ref, out_ref,
             stage1_ref, comm1_ref, stage2_ref, comm2_ref,
             send1_sems, recv1_sems, send2_sems, recv2_sems):
        my = lax.axis_index("i")

        xf = x_ref[:, :]
        scores = jnp.dot(xf, rw_ref[:, :], preferred_element_type=jnp.float32)
        e_iota = lax.broadcasted_iota(jnp.int32, (N_TOK, N_EXP), 1)
        r0 = idx_ref[:, 0:1]
        r1 = idx_ref[:, 1:2]
        s0 = jnp.sum(jnp.where(e_iota == r0, scores, 0.0), axis=1, keepdims=True)
        s1 = jnp.sum(jnp.where(e_iota == r1, scores, 0.0), axis=1, keepdims=True)
        w0 = 1.0 / (1.0 + jnp.exp(s1 - s0))
        w1 = 1.0 - w0

        le = my * E_LOCAL + lax.broadcasted_iota(jnp.int32, (1, E_LOCAL), 1)
        wloc = (w0 * (r0 == le).astype(jnp.float32)
                + w1 * (r1 == le).astype(jnp.float32))
        partial = jnp.zeros((N_TOK, D_OUT), jnp.float32)
        for k in range(E_LOCAL):
            xk = (xf * wloc[:, k:k + 1]).astype(jnp.bfloat16)
            wk = ew_ref[k, :, :].astype(jnp.bfloat16)
            partial = partial + jnp.dot(xk, wk, preferred_element_type=jnp.float32)
        stage1_ref[:, :] = partial.astype(jnp.bfloat16)

        sends1 = []
        for d in range(1, N_DEV):
            dst = lax.rem(my + d, N_DEV)
            r = pltpu.make_async_remote_copy(
                src_ref=stage1_ref.at[pl.ds(dst * ROWS, ROWS), :],
                dst_ref=comm1_ref.at[my],
                send_sem=send1_sems.at[d],
                recv_sem=recv1_sems.at[my],
                device_id=(dst,),
                device_id_type=pl.DeviceIdType.MESH,
            )
            r.start()
            sends1.append(r)
        comm1_ref[pl.ds(my, 1)] = stage1_ref[pl.ds(my * ROWS, ROWS), :].reshape(
            1, ROWS, D_OUT)
        for d in range(1, N_DEV):
            src = lax.rem(my - d + N_DEV, N_DEV)
            rr = pltpu.make_async_remote_copy(
                src_ref=stage1_ref.at[pl.ds(0, ROWS), :],
                dst_ref=comm1_ref.at[src],
                send_sem=send1_sems.at[d],
                recv_sem=recv1_sems.at[src],
                device_id=(src,),
                device_id_type=pl.DeviceIdType.MESH,
            )
            rr.wait_recv()
        reduced = jnp.sum(comm1_ref[:, :, :].astype(jnp.float32), axis=0)
        for r in sends1:
            r.wait_send()

        stage2_ref[:, :] = reduced.astype(jnp.bfloat16)
        sends2 = []
        for d in range(1, N_DEV):
            dst = lax.rem(my + d, N_DEV)
            r = pltpu.make_async_remote_copy(
                src_ref=stage2_ref,
                dst_ref=comm2_ref.at[my],
                send_sem=send2_sems.at[d],
                recv_sem=recv2_sems.at[my],
                device_id=(dst,),
                device_id_type=pl.DeviceIdType.MESH,
            )
            r.start()
            sends2.append(r)
        comm2_ref[pl.ds(my, 1)] = stage2_ref[:, :].reshape(1, ROWS, D_OUT)
        for d in range(1, N_DEV):
            src = lax.rem(my - d + N_DEV, N_DEV)
            rr = pltpu.make_async_remote_copy(
                src_ref=stage2_ref,
                dst_ref=comm2_ref.at[src],
                send_sem=send2_sems.at[d],
                recv_sem=recv2_sems.at[src],
                device_id=(src,),
                device_id_type=pl.DeviceIdType.MESH,
            )
            rr.wait_recv()
        for r in sends2:
            r.wait_send()
        out_ref[:, :] = comm2_ref[:, :, :].astype(jnp.float32).reshape(N_TOK, D_OUT)

    return pl.pallas_call(
        body,
        out_shape=jax.ShapeDtypeStruct((N_TOK, D_OUT), jnp.float32),
        in_specs=[pl.BlockSpec(memory_space=pltpu.VMEM)] * 4,
        out_specs=pl.BlockSpec(memory_space=pltpu.VMEM),
        scratch_shapes=[
            pltpu.VMEM((N_TOK, D_OUT), jnp.bfloat16),
            pltpu.VMEM((N_DEV, ROWS, D_OUT), jnp.bfloat16),
            pltpu.VMEM((ROWS, D_OUT), jnp.bfloat16),
            pltpu.VMEM((N_DEV, ROWS, D_OUT), jnp.bfloat16),
            pltpu.SemaphoreType.DMA((N_DEV,)),
            pltpu.SemaphoreType.DMA((N_DEV,)),
            pltpu.SemaphoreType.DMA((N_DEV,)),
            pltpu.SemaphoreType.DMA((N_DEV,)),
        ],
    )(x, router_W, route_idx, expert_W)


# device time: 30890 ns/iter; 1.2048x vs baseline; 1.2048x over previous
import jax
import jax.numpy as jnp
from jax import lax
from jax.experimental import pallas as pl
from jax.experimental.pallas import tpu as pltpu

N_DEV = 32
N_TOK = 512
D_IN = 256
D_OUT = 512
N_EXP = 128
E_LOCAL = N_EXP // N_DEV
ROWS = N_TOK // N_DEV


def kernel(x, router_W, route_idx, expert_W):
    def body(x_ref, rw_ref, idx_ref, ew_ref, out_ref,
             stage1_ref, comm1_ref, stage2_ref, comm2_ref,
             send1_sems, recv1_sems, send2_sems, recv2_sems):
        my = lax.axis_index("i")

        barrier_sem = pltpu.get_barrier_semaphore()
        for d in range(1, N_DEV):
            pl.semaphore_signal(
                barrier_sem, inc=1,
                device_id=(lax.rem(my + d, N_DEV),),
                device_id_type=pl.DeviceIdType.MESH,
            )

        xf = x_ref[:, :]
        scores = jnp.dot(xf, rw_ref[:, :], preferred_element_type=jnp.float32)
        e_iota = lax.broadcasted_iota(jnp.int32, (N_TOK, N_EXP), 1)
        r0 = idx_ref[:, 0:1]
        r1 = idx_ref[:, 1:2]
        s0 = jnp.sum(jnp.where(e_iota == r0, scores, 0.0), axis=1, keepdims=True)
        s1 = jnp.sum(jnp.where(e_iota == r1, scores, 0.0), axis=1, keepdims=True)
        w0 = 1.0 / (1.0 + jnp.exp(s1 - s0))
        w1 = 1.0 - w0

        le = my * E_LOCAL + lax.broadcasted_iota(jnp.int32, (1, E_LOCAL), 1)
        wloc = (w0 * (r0 == le).astype(jnp.float32)
                + w1 * (r1 == le).astype(jnp.float32))

        BLK = 128
        CHUNKS_PER_BLK = BLK // ROWS
        sends1 = []
        for b in range(N_TOK // BLK):
            rows = pl.ds(b * BLK, BLK)
            xb = xf[b * BLK:(b + 1) * BLK, :]
            pblk = jnp.zeros((BLK, D_OUT), jnp.float32)
            for k in range(E_LOCAL):
                xk = (xb * wloc[b * BLK:(b + 1) * BLK, k:k + 1]).astype(jnp.bfloat16)
                wk = ew_ref[k, :, :].astype(jnp.bfloat16)
                pblk = pblk + jnp.dot(xk, wk, preferred_element_type=jnp.float32)
            stage1_ref[rows, :] = pblk.astype(jnp.bfloat16)
            if b == 0:
                pl.semaphore_wait(barrier_sem, N_DEV - 1)
            for j in range(b * CHUNKS_PER_BLK, (b + 1) * CHUNKS_PER_BLK):
                r = pltpu.make_async_remote_copy(
                    src_ref=stage1_ref.at[pl.ds(j * ROWS, ROWS), :],
                    dst_ref=comm1_ref.at[my],
                    send_sem=send1_sems.at[j],
                    recv_sem=recv1_sems.at[my],
                    device_id=(j,),
                    device_id_type=pl.DeviceIdType.MESH,
                )

                @pl.when(j != my)
                def _(r=r):
                    r.start()

                sends1.append((j, r))
        comm1_ref[pl.ds(my, 1)] = stage1_ref[pl.ds(my * ROWS, ROWS), :].reshape(
            1, ROWS, D_OUT)
        for d in range(1, N_DEV):
            src = lax.rem(my - d + N_DEV, N_DEV)
            rr = pltpu.make_async_remote_copy(
                src_ref=stage1_ref.at[pl.ds(0, ROWS), :],
                dst_ref=comm1_ref.at[src],
                send_sem=send1_sems.at[d],
                recv_sem=recv1_sems.at[src],
                device_id=(src,),
                device_id_type=pl.DeviceIdType.MESH,
            )
            rr.wait_recv()
        reduced = jnp.sum(comm1_ref[:, :, :].astype(jnp.float32), axis=0)
        for j, r in sends1:
            @pl.when(j != my)
            def _(r=r):
                r.wait_send()

        stage2_ref[:, :] = reduced.astype(jnp.bfloat16)
        sends2 = []
        for d in range(1, N_DEV):
            dst = lax.rem(my + d, N_DEV)
            r = pltpu.make_async_remote_copy(
                src_ref=stage2_ref,
                dst_ref=comm2_ref.at[my],
                send_sem=send2_sems.at[d],
                recv_sem=recv2_sems.at[my],
                device_id=(dst,),
                device_id_type=pl.DeviceIdType.MESH,
            )
            r.start()
            sends2.append(r)
        comm2_ref[pl.ds(my, 1)] = stage2_ref[:, :].reshape(1, ROWS, D_OUT)
        for d in range(1, N_DEV):
            src = lax.rem(my - d + N_DEV, N_DEV)
            rr = pltpu.make_async_remote_copy(
                src_ref=stage2_ref,
                dst_ref=comm2_ref.at[src],
                send_sem=send2_sems.at[d],
                recv_sem=recv2_sems.at[src],
                device_id=(src,),
                device_id_type=pl.DeviceIdType.MESH,
            )
            rr.wait_recv()
        for r in sends2:
            r.wait_send()
        out_ref[:, :] = comm2_ref[:, :, :].astype(jnp.float32).reshape(N_TOK, D_OUT)

    return pl.pallas_call(
        body,
        out_shape=jax.ShapeDtypeStruct((N_TOK, D_OUT), jnp.float32),
        in_specs=[pl.BlockSpec(memory_space=pltpu.VMEM)] * 4,
        out_specs=pl.BlockSpec(memory_space=pltpu.VMEM),
        scratch_shapes=[
            pltpu.VMEM((N_TOK, D_OUT), jnp.bfloat16),
            pltpu.VMEM((N_DEV, ROWS, D_OUT), jnp.bfloat16),
            pltpu.VMEM((ROWS, D_OUT), jnp.bfloat16),
            pltpu.VMEM((N_DEV, ROWS, D_OUT), jnp.bfloat16),
            pltpu.SemaphoreType.DMA((N_DEV,)),
            pltpu.SemaphoreType.DMA((N_DEV,)),
            pltpu.SemaphoreType.DMA((N_DEV,)),
            pltpu.SemaphoreType.DMA((N_DEV,)),
        ],
        compiler_params=pltpu.CompilerParams(collective_id=0),
    )(x, router_W, route_idx, expert_W)


# device time: 30357 ns/iter; 1.2260x vs baseline; 1.0176x over previous
import jax
import jax.numpy as jnp
from jax import lax
from jax.experimental import pallas as pl
from jax.experimental.pallas import tpu as pltpu

N_DEV = 32
N_TOK = 512
D_IN = 256
D_OUT = 512
N_EXP = 128
E_LOCAL = N_EXP // N_DEV
ROWS = N_TOK // N_DEV
BLK = 128
CHUNKS_PER_BLK = BLK // ROWS


def kernel(x, router_W, route_idx, expert_W):
    def body(x_ref, rw_ref, idx_ref, ew_ref, out_ref,
             ewb_ref, stage1_ref, comm1_ref, stage2_ref, comm2_ref,
             send1_sems, recv1_sems, send2_sems, recv2_sems):
        my = lax.axis_index("i")

        barrier_sem = pltpu.get_barrier_semaphore()
        for j in range(N_DEV):
            @pl.when(j != my)
            def _(j=j):
                pl.semaphore_signal(
                    barrier_sem, inc=1,
                    device_id=(j,),
                    device_id_type=pl.DeviceIdType.MESH,
                )

        for k in range(E_LOCAL):
            ewb_ref[:, k * D_OUT:(k + 1) * D_OUT] = ew_ref[k, :, :].astype(
                jnp.bfloat16)

        xf = x_ref[:, :]
        scores = jnp.dot(xf, rw_ref[:, :], preferred_element_type=jnp.float32)
        e_iota = lax.broadcasted_iota(jnp.int32, (N_TOK, N_EXP), 1)
        r0 = idx_ref[:, 0:1]
        r1 = idx_ref[:, 1:2]
        s0 = jnp.sum(jnp.where(e_iota == r0, scores, 0.0), axis=1, keepdims=True)
        s1 = jnp.sum(jnp.where(e_iota == r1, scores, 0.0), axis=1, keepdims=True)
        w0 = 1.0 / (1.0 + jnp.exp(s1 - s0))
        w1 = 1.0 - w0
        le = my * E_LOCAL + lax.broadcasted_iota(jnp.int32, (1, E_LOCAL), 1)
        wloc = (w0 * (r0 == le).astype(jnp.float32)
                + w1 * (r1 == le).astype(jnp.float32))

        sends1 = []
        for b in range(N_TOK // BLK):
            lo = b * BLK
            xb = xf[lo:lo + BLK, :].astype(jnp.bfloat16)
            y = jnp.dot(xb, ewb_ref[:, :], preferred_element_type=jnp.float32)
            pblk = jnp.zeros((BLK, D_OUT), jnp.float32)
            for k in range(E_LOCAL):
                pblk = pblk + wloc[lo:lo + BLK, k:k + 1] * y[:, k * D_OUT:(k + 1) * D_OUT]
            stage1_ref[pl.ds(lo, BLK), :] = pblk.astype(jnp.bfloat16)
            if b == 0:
                pl.semaphore_wait(barrier_sem, N_DEV - 1)
            for j in range(b * CHUNKS_PER_BLK, (b + 1) * CHUNKS_PER_BLK):
                r = pltpu.make_async_remote_copy(
                    src_ref=stage1_ref.at[pl.ds(j * ROWS, ROWS), :],
                    dst_ref=comm1_ref.at[my],
                    send_sem=send1_sems.at[j],
                    recv_sem=recv1_sems.at[my],
                    device_id=(j,),
                    device_id_type=pl.DeviceIdType.MESH,
                )

                @pl.when(j != my)
                def _(r=r):
                    r.start()

                sends1.append((j, r))
        comm1_ref[pl.ds(my, 1)] = stage1_ref[pl.ds(my * ROWS, ROWS), :].reshape(
            1, ROWS, D_OUT)
        for j in range(N_DEV):
            rr = pltpu.make_async_remote_copy(
                src_ref=stage1_ref.at[pl.ds(0, ROWS), :],
                dst_ref=comm1_ref.at[j],
                send_sem=send1_sems.at[j],
                recv_sem=recv1_sems.at[j],
                device_id=(j,),
                device_id_type=pl.DeviceIdType.MESH,
            )

            @pl.when(j != my)
            def _(rr=rr):
                rr.wait_recv()
        reduced = jnp.sum(comm1_ref[:, :, :].astype(jnp.float32), axis=0)
        for j, r in sends1:
            @pl.when(j != my)
            def _(r=r):
                r.wait_send()

        stage2_ref[:, :] = reduced.astype(jnp.bfloat16)
        sends2 = []
        for j in range(N_DEV):
            r = pltpu.make_async_remote_copy(
                src_ref=stage2_ref,
                dst_ref=comm2_ref.at[my],
                send_sem=send2_sems.at[j],
                recv_sem=recv2_sems.at[my],
                device_id=(j,),
                device_id_type=pl.DeviceIdType.MESH,
            )

            @pl.when(j != my)
            def _(r=r):
                r.start()

            sends2.append((j, r))
        comm2_ref[pl.ds(my, 1)] = stage2_ref[:, :].reshape(1, ROWS, D_OUT)
        for j in range(N_DEV):
            rr = pltpu.make_async_remote_copy(
                src_ref=stage2_ref,
                dst_ref=comm2_ref.at[j],
                send_sem=send2_sems.at[j],
                recv_sem=recv2_sems.at[j],
                device_id=(j,),
                device_id_type=pl.DeviceIdType.MESH,
            )

            @pl.when(j != my)
            def _(rr=rr):
                rr.wait_recv()
        for j, r in sends2:
            @pl.when(j != my)
            def _(r=r):
                r.wait_send()
        out_ref[:, :] = comm2_ref[:, :, :].astype(jnp.float32).reshape(N_TOK, D_OUT)

    return pl.pallas_call(
        body,
        out_shape=jax.ShapeDtypeStruct((N_TOK, D_OUT), jnp.float32),
        in_specs=[pl.BlockSpec(memory_space=pltpu.VMEM)] * 4,
        out_specs=pl.BlockSpec(memory_space=pltpu.VMEM),
        scratch_shapes=[
            pltpu.VMEM((D_IN, E_LOCAL * D_OUT), jnp.bfloat16),
            pltpu.VMEM((N_TOK, D_OUT), jnp.bfloat16),
            pltpu.VMEM((N_DEV, ROWS, D_OUT), jnp.bfloat16),
            pltpu.VMEM((ROWS, D_OUT), jnp.bfloat16),
            pltpu.VMEM((N_DEV, ROWS, D_OUT), jnp.bfloat16),
            pltpu.SemaphoreType.DMA((N_DEV,)),
            pltpu.SemaphoreType.DMA((N_DEV,)),
            pltpu.SemaphoreType.DMA((N_DEV,)),
            pltpu.SemaphoreType.DMA((N_DEV,)),
        ],
        compiler_params=pltpu.CompilerParams(collective_id=0),
    )(x, router_W, route_idx, expert_W)
